# baseline (device time: 26503 ns/iter reference)
import jax
import jax.numpy as jnp
from jax import lax
from jax.experimental import pallas as pl
from jax.experimental.pallas import tpu as pltpu

K = 16


def _topk_desc(v, k):
    cols = []
    for i in range(k):
        mi = jnp.max(v, axis=1, keepdims=True)
        cols.append(mi)
        if i < k - 1:
            v = jnp.where(v == mi, -jnp.inf, v)
    return jnp.concatenate(cols, axis=1)


def kernel(x):
    m, n_loc = x.shape
    n_half = n_loc // 2
    n_chunk = n_half // 2

    def body(
        x_hbm,
        o_ref,
        xa_ref,
        xb_ref,
        ca_ref,
        cb_ref,
        recv_ref,
        dma_sems,
        send_sems,
        recv_sems,
    ):
        my_x = lax.axis_index("x")
        my_y = lax.axis_index("y")
        peers = [
            (my_x, 1 - my_y),
            (1 - my_x, my_y),
            (1 - my_x, 1 - my_y),
        ]

        barrier = pltpu.get_barrier_semaphore()
        for p in peers:
            pl.semaphore_signal(
                barrier, inc=1, device_id=p,
                device_id_type=pl.DeviceIdType.MESH,
            )

        base = my_y * n_half
        dma0 = pltpu.make_async_copy(
            x_hbm.at[:, pl.ds(base, n_chunk)], xa_ref, dma_sems.at[0]
        )
        dma0.start()
        dma1 = pltpu.make_async_copy(
            x_hbm.at[:, pl.ds(base + n_chunk, n_chunk)], xb_ref, dma_sems.at[1]
        )
        dma1.start()

        dma0.wait()
        ca_ref[:, :] = _topk_desc(xa_ref[:, :], K)

        pl.semaphore_wait(barrier, 3)

        def broadcast(src_ref, batch):
            rs = []
            for i, p in enumerate(peers):
                r = pltpu.make_async_remote_copy(
                    src_ref=src_ref,
                    dst_ref=recv_ref.at[batch, i],
                    send_sem=send_sems.at[batch * 3 + i],
                    recv_sem=recv_sems.at[batch * 3 + i],
                    device_id=p,
                    device_id_type=pl.DeviceIdType.MESH,
                )
                r.start()
                rs.append(r)
            return rs

        rdmas_a = broadcast(ca_ref, 0)

        dma1.wait()
        cb_ref[:, :] = _topk_desc(xb_ref[:, :], K)
        rdmas_b = broadcast(cb_ref, 1)

        for r in rdmas_a + rdmas_b:
            r.wait()

        allc = jnp.concatenate(
            [
                ca_ref[:, :],
                cb_ref[:, :],
                recv_ref[0, 0],
                recv_ref[0, 1],
                recv_ref[0, 2],
                recv_ref[1, 0],
                recv_ref[1, 1],
                recv_ref[1, 2],
            ],
            axis=1,
        )
        o_ref[:, :] = _topk_desc(allc, K)

    return pl.pallas_call(
        body,
        out_shape=jax.ShapeDtypeStruct((m, K), jnp.float32),
        in_specs=[pl.BlockSpec(memory_space=pltpu.MemorySpace.HBM)],
        out_specs=pl.BlockSpec(memory_space=pltpu.VMEM),
        scratch_shapes=[
            pltpu.VMEM((m, n_chunk), jnp.float32),
            pltpu.VMEM((m, n_chunk), jnp.float32),
            pltpu.VMEM((m, K), jnp.float32),
            pltpu.VMEM((m, K), jnp.float32),
            pltpu.VMEM((2, 3, m, K), jnp.float32),
            pltpu.SemaphoreType.DMA((2,)),
            pltpu.SemaphoreType.DMA((6,)),
            pltpu.SemaphoreType.DMA((6,)),
        ],
        compiler_params=pltpu.CompilerParams(collective_id=0),
    )(x)


# device time: 22429 ns/iter; 1.1816x vs baseline; 1.1816x over previous
import jax
import jax.numpy as jnp
from jax import lax
from jax.experimental import pallas as pl
from jax.experimental.pallas import tpu as pltpu

K = 16


def _topk_desc(v, k):
    m = jnp.max(v, axis=1, keepdims=True)
    cols = [m]
    for _ in range(k - 1):
        m = jnp.max(jnp.where(v < m, v, -jnp.inf), axis=1, keepdims=True)
        cols.append(m)
    return jnp.concatenate(cols, axis=1)


def kernel(x):
    m, n_loc = x.shape
    n_half = n_loc // 2

    def body(x_ref, o_ref, a_ref, recv_ref, send_sems, recv_sems):
        my_x = lax.axis_index("x")
        my_y = lax.axis_index("y")
        peers = [
            (my_x, 1 - my_y),
            (1 - my_x, my_y),
            (1 - my_x, 1 - my_y),
        ]

        barrier = pltpu.get_barrier_semaphore()
        for p in peers:
            pl.semaphore_signal(
                barrier, inc=1, device_id=p,
                device_id_type=pl.DeviceIdType.MESH,
            )

        a_ref[:, :] = _topk_desc(x_ref[:, pl.ds(my_y * n_half, n_half)], K)

        pl.semaphore_wait(barrier, 3)

        rdmas = []
        for i, p in enumerate(peers):
            r = pltpu.make_async_remote_copy(
                src_ref=a_ref,
                dst_ref=recv_ref.at[i],
                send_sem=send_sems.at[i],
                recv_sem=recv_sems.at[i],
                device_id=p,
                device_id_type=pl.DeviceIdType.MESH,
            )
            r.start()
            rdmas.append(r)
        for r in rdmas:
            r.wait()

        allc = jnp.concatenate(
            [a_ref[:, :], recv_ref[0], recv_ref[1], recv_ref[2]], axis=1
        )
        o_ref[:, :] = _topk_desc(allc, K)

    return pl.pallas_call(
        body,
        out_shape=jax.ShapeDtypeStruct((m, K), jnp.float32),
        in_specs=[pl.BlockSpec(memory_space=pltpu.VMEM)],
        out_specs=pl.BlockSpec(memory_space=pltpu.VMEM),
        scratch_shapes=[
            pltpu.VMEM((m, K), jnp.float32),
            pltpu.VMEM((3, m, K), jnp.float32),
            pltpu.SemaphoreType.DMA((3,)),
            pltpu.SemaphoreType.DMA((3,)),
        ],
        compiler_params=pltpu.CompilerParams(collective_id=0),
    )(x)


# device time: 20051 ns/iter; 1.3218x vs baseline; 1.1186x over previous
import jax
import jax.numpy as jnp
from jax import lax
from jax.experimental import pallas as pl
from jax.experimental.pallas import tpu as pltpu

K = 16


def _topk_desc(v, k):
    m = jnp.max(v, axis=1, keepdims=True)
    cols = [m]
    for _ in range(k - 1):
        m = jnp.max(jnp.where(v < m, v, -jnp.inf), axis=1, keepdims=True)
        cols.append(m)
    return jnp.concatenate(cols, axis=1)


def kernel(x):
    m, n_loc = x.shape
    m_half = m // 2

    def body(x_hbm, o_ref, xv_ref, a_ref, recv_ref, dma_sem, send_sems, recv_sems):
        my_x = lax.axis_index("x")
        my_y = lax.axis_index("y")
        peers = [
            (my_x, 1 - my_y),
            (1 - my_x, my_y),
            (1 - my_x, 1 - my_y),
        ]

        barrier = pltpu.get_barrier_semaphore()
        for p in peers:
            pl.semaphore_signal(
                barrier, inc=1, device_id=p,
                device_id_type=pl.DeviceIdType.MESH,
            )

        dma = pltpu.make_async_copy(
            x_hbm.at[pl.ds(my_y * m_half, m_half), :], xv_ref, dma_sem
        )
        dma.start()
        dma.wait()

        a_ref[:, :] = _topk_desc(xv_ref[:, :], K)

        pl.semaphore_wait(barrier, 3)

        rdmas = [None, None, None]
        for i in (2, 1, 0):
            r = pltpu.make_async_remote_copy(
                src_ref=a_ref,
                dst_ref=recv_ref.at[i],
                send_sem=send_sems.at[i],
                recv_sem=recv_sems.at[i],
                device_id=peers[i],
                device_id_type=pl.DeviceIdType.MESH,
            )
            r.start()
            rdmas[i] = r

        rdmas[1].wait_recv()
        o_ref[pl.ds(my_y * m_half, m_half), :] = _topk_desc(
            jnp.concatenate([a_ref[:, :], recv_ref[1]], axis=1), K
        )

        rdmas[0].wait_recv()
        rdmas[2].wait_recv()
        o_ref[pl.ds((1 - my_y) * m_half, m_half), :] = _topk_desc(
            jnp.concatenate([recv_ref[0], recv_ref[2]], axis=1), K
        )

        for r in rdmas:
            r.wait_send()

    return pl.pallas_call(
        body,
        out_shape=jax.ShapeDtypeStruct((m, K), jnp.float32),
        in_specs=[pl.BlockSpec(memory_space=pltpu.MemorySpace.HBM)],
        out_specs=pl.BlockSpec(memory_space=pltpu.VMEM),
        scratch_shapes=[
            pltpu.VMEM((m_half, n_loc), jnp.float32),
            pltpu.VMEM((m_half, K), jnp.float32),
            pltpu.VMEM((3, m_half, K), jnp.float32),
            pltpu.SemaphoreType.DMA,
            pltpu.SemaphoreType.DMA((3,)),
            pltpu.SemaphoreType.DMA((3,)),
        ],
        compiler_params=pltpu.CompilerParams(collective_id=0),
    )(x)


# device time: 19922 ns/iter; 1.3303x vs baseline; 1.0065x over previous
import jax
import jax.numpy as jnp
from jax import lax
from jax.experimental import pallas as pl
from jax.experimental.pallas import tpu as pltpu

K = 16


def _topk_desc(v, k):
    n = v.shape[1]
    hi = jnp.maximum(v[:, : n // 2], v[:, n // 2 :])
    lo = jnp.minimum(v[:, : n // 2], v[:, n // 2 :])
    cols = []
    m = None
    for _ in range(k // 2):
        if m is None:
            whi, wlo = hi, lo
        else:
            whi = jnp.where(hi < m, hi, -jnp.inf)
            wlo = jnp.where(lo < m, lo, -jnp.inf)
        cur = jnp.maximum(whi, wlo)
        sec = jnp.minimum(whi, wlo)
        m1 = jnp.max(cur, axis=1, keepdims=True)
        cand = jnp.where(cur == m1, sec, cur)
        m2 = jnp.max(cand, axis=1, keepdims=True)
        cols.append(m1)
        cols.append(m2)
        m = m2
    return jnp.concatenate(cols, axis=1)


def kernel(x):
    m, n_loc = x.shape
    m_half = m // 2

    def body(x_hbm, o_ref, xv_ref, a_ref, recv_ref, dma_sem, send_sems, recv_sems):
        my_x = lax.axis_index("x")
        my_y = lax.axis_index("y")
        peers = [
            (my_x, 1 - my_y),
            (1 - my_x, my_y),
            (1 - my_x, 1 - my_y),
        ]

        barrier = pltpu.get_barrier_semaphore()
        for p in peers:
            pl.semaphore_signal(
                barrier, inc=1, device_id=p,
                device_id_type=pl.DeviceIdType.MESH,
            )

        dma = pltpu.make_async_copy(
            x_hbm.at[pl.ds(my_y * m_half, m_half), :], xv_ref, dma_sem
        )
        dma.start()
        dma.wait()

        a_ref[:, :] = _topk_desc(xv_ref[:, :], K)

        pl.semaphore_wait(barrier, 3)

        rdmas = [None, None, None]
        for i in (2, 1, 0):
            r = pltpu.make_async_remote_copy(
                src_ref=a_ref,
                dst_ref=recv_ref.at[i],
                send_sem=send_sems.at[i],
                recv_sem=recv_sems.at[i],
                device_id=peers[i],
                device_id_type=pl.DeviceIdType.MESH,
            )
            r.start()
            rdmas[i] = r

        rdmas[1].wait_recv()
        o_ref[pl.ds(my_y * m_half, m_half), :] = _topk_desc(
            jnp.concatenate([a_ref[:, :], recv_ref[1]], axis=1), K
        )

        rdmas[0].wait_recv()
        rdmas[2].wait_recv()
        o_ref[pl.ds((1 - my_y) * m_half, m_half), :] = _topk_desc(
            jnp.concatenate([recv_ref[0], recv_ref[2]], axis=1), K
        )

        for r in rdmas:
            r.wait_send()

    return pl.pallas_call(
        body,
        out_shape=jax.ShapeDtypeStruct((m, K), jnp.float32),
        in_specs=[pl.BlockSpec(memory_space=pltpu.MemorySpace.HBM)],
        out_specs=pl.BlockSpec(memory_space=pltpu.VMEM),
        scratch_shapes=[
            pltpu.VMEM((m_half, n_loc), jnp.float32),
            pltpu.VMEM((m_half, K), jnp.float32),
            pltpu.VMEM((3, m_half, K), jnp.float32),
            pltpu.SemaphoreType.DMA,
            pltpu.SemaphoreType.DMA((3,)),
            pltpu.SemaphoreType.DMA((3,)),
        ],
        compiler_params=pltpu.CompilerParams(collective_id=0),
    )(x)


# device time: 19615 ns/iter; 1.3512x vs baseline; 1.0157x over previous
import jax
import jax.numpy as jnp
from jax import lax
from jax.experimental import pallas as pl
from jax.experimental.pallas import tpu as pltpu

K = 16


def _topk_desc(v, k):
    n = v.shape[1]
    return _topk_desc_pair(v[:, : n // 2], v[:, n // 2 :], k)


def _topk_desc_pair(va, vb, k):
    hi = jnp.maximum(va, vb)
    lo = jnp.minimum(va, vb)
    cols = []
    m = None
    for _ in range(k // 2):
        if m is None:
            whi, wlo = hi, lo
        else:
            whi = jnp.where(hi < m, hi, -jnp.inf)
            wlo = jnp.where(lo < m, lo, -jnp.inf)
        cur = jnp.maximum(whi, wlo)
        sec = jnp.minimum(whi, wlo)
        m1 = jnp.max(cur, axis=1, keepdims=True)
        cand = jnp.where(cur == m1, sec, cur)
        m2 = jnp.max(cand, axis=1, keepdims=True)
        cols.append(m1)
        cols.append(m2)
        m = m2
    return jnp.concatenate(cols, axis=1)


def kernel(x):
    m, n_loc = x.shape
    m_half = m // 2

    def body(x_hbm, o_ref, xv_ref, a_ref, recv_ref, dma_sem, send_sems, recv_sems):
        my_x = lax.axis_index("x")
        my_y = lax.axis_index("y")
        peers = [
            (my_x, 1 - my_y),
            (1 - my_x, my_y),
            (1 - my_x, 1 - my_y),
        ]

        barrier = pltpu.get_barrier_semaphore()
        for p in peers:
            pl.semaphore_signal(
                barrier, inc=1, device_id=p,
                device_id_type=pl.DeviceIdType.MESH,
            )

        dma = pltpu.make_async_copy(
            x_hbm.at[pl.ds(my_y * m_half, m_half), :], xv_ref, dma_sem
        )
        dma.start()
        dma.wait()

        a_ref[:, :] = _topk_desc(xv_ref[:, :], K)

        pl.semaphore_wait(barrier, 3)

        rdmas = [None, None, None]
        for i in (2, 1, 0):
            r = pltpu.make_async_remote_copy(
                src_ref=a_ref,
                dst_ref=recv_ref.at[i],
                send_sem=send_sems.at[i],
                recv_sem=recv_sems.at[i],
                device_id=peers[i],
                device_id_type=pl.DeviceIdType.MESH,
            )
            r.start()
            rdmas[i] = r

        rdmas[1].wait_recv()
        o_ref[pl.ds(my_y * m_half, m_half), :] = _topk_desc_pair(
            a_ref[:, :], recv_ref[1], K
        )

        rdmas[0].wait_recv()
        rdmas[2].wait_recv()
        o_ref[pl.ds((1 - my_y) * m_half, m_half), :] = _topk_desc_pair(
            recv_ref[0], recv_ref[2], K
        )

        for r in rdmas:
            r.wait_send()

    return pl.pallas_call(
        body,
        out_shape=jax.ShapeDtypeStruct((m, K), jnp.float32),
        in_specs=[pl.BlockSpec(memory_space=pltpu.MemorySpace.HBM)],
        out_specs=pl.BlockSpec(memory_space=pltpu.VMEM),
        scratch_shapes=[
            pltpu.VMEM((m_half, n_loc), jnp.float32),
            pltpu.VMEM((m_half, K), jnp.float32),
            pltpu.VMEM((3, m_half, K), jnp.float32),
            pltpu.SemaphoreType.DMA,
            pltpu.SemaphoreType.DMA((3,)),
            pltpu.SemaphoreType.DMA((3,)),
        ],
        compiler_params=pltpu.CompilerParams(collective_id=0),
    )(x)
